# baseline (speedup 1.0000x reference)
import functools

import numpy as np
import jax
import jax.numpy as jnp
from jax import lax
from jax.experimental import pallas as pl
from jax.experimental.pallas import tpu as pltpu
from jax.experimental.pallas import tpu_sc as plsc

TIN = 2048
FIN = 256
TSEQ = 2072
NF = TSEQ // 2 + 1
NFP = 1040
D = 768
H = 12
DH = D // H
DFF = 3072
TOPK = 32
KDEC = 25
PADC = (KDEC - 1) // 2
RB = 296
KB = 208

_F32 = jnp.float32

def _dft_consts():
    t = np.arange(TSEQ, dtype=np.int64)
    k = np.arange(NFP, dtype=np.int64)
    m = (k[:, None] * t[None, :]) % TSEQ
    ang = (2.0 * np.pi / TSEQ) * m
    cc = np.cos(ang)
    cs = -np.sin(ang)
    cc[NF:] = 0.0
    cs[NF:] = 0.0
    return cc.astype(np.float32), cs.astype(np.float32)


def _pe_const():
    pos = np.arange(TSEQ, dtype=np.float64)[:, None]
    div = np.exp(np.arange(0, D, 2, dtype=np.float64) * (-np.log(10000.0) / D))
    pe = np.zeros((TSEQ, D), np.float64)
    pe[:, 0::2] = np.sin(pos * div)
    pe[:, 1::2] = np.cos(pos * div)
    return pe.astype(np.float32)


_CCOS_NP, _CSIN_NP = _dft_consts()
_PE_NP = _pe_const()


def _split_bf16(a):
    hi = a.astype(np.float32).astype(jnp.bfloat16)
    lo = (a - np.asarray(hi, np.float32)).astype(jnp.bfloat16)
    return np.asarray(hi), np.asarray(lo)


_CCOS_HI, _CCOS_LO = _split_bf16(_CCOS_NP)
_CSIN_HI, _CSIN_LO = _split_bf16(_CSIN_NP)

_HI = lax.Precision.HIGHEST


def _dg11(a, b, precision=None):
    return lax.dot_general(a, b, (((1,), (1,)), ((), ())),
                           precision=precision, preferred_element_type=_F32)


def _dg10(a, b, precision=None):
    return lax.dot_general(a, b, (((1,), (0,)), ((), ())),
                           precision=precision, preferred_element_type=_F32)


def _embed(xp, w, b, pe):
    def body(xp_ref, w_ref, b_ref, pe_ref, o_ref):
        acc = xp_ref[0:TSEQ, :]
        for j in range(1, KDEC):
            acc = acc + xp_ref[j:j + TSEQ, :]
        seasonal = xp_ref[PADC:PADC + TSEQ, :] - acc * (1.0 / KDEC)
        h = _dg11(seasonal, w_ref[...])
        o_ref[...] = h + b_ref[...] + pe_ref[...]

    return pl.pallas_call(
        body,
        out_shape=jax.ShapeDtypeStruct((TSEQ, D), _F32),
    )(xp, w, b, pe)


def _spectrum(h, cchi, cclo, cshi, cslo):
    def body(h_ref, cchi_ref, cclo_ref, cshi_ref, cslo_ref, re_ref, amp_ref):
        h_ = h_ref[...]
        h_hi = h_.astype(jnp.bfloat16)
        h_lo = (h_ - h_hi.astype(_F32)).astype(jnp.bfloat16)
        re = (_dg10(cchi_ref[...], h_hi) + _dg10(cchi_ref[...], h_lo)
              + _dg10(cclo_ref[...], h_hi))
        im = (_dg10(cshi_ref[...], h_hi) + _dg10(cshi_ref[...], h_lo)
              + _dg10(cslo_ref[...], h_hi))
        re_ref[...] = re
        amp = jnp.sqrt(re * re + im * im).mean(axis=1)
        i = pl.program_id(0)
        row = i * KB + lax.broadcasted_iota(jnp.int32, (1, 1, KB), 2)
        amp_ref[...] = jnp.where(row < NF, amp[None, None, :], -1.0)

    re, amp = pl.pallas_call(
        body,
        grid=(NFP // KB,),
        in_specs=[
            pl.BlockSpec((TSEQ, D), lambda i: (0, 0)),
            pl.BlockSpec((KB, TSEQ), lambda i: (i, 0)),
            pl.BlockSpec((KB, TSEQ), lambda i: (i, 0)),
            pl.BlockSpec((KB, TSEQ), lambda i: (i, 0)),
            pl.BlockSpec((KB, TSEQ), lambda i: (i, 0)),
        ],
        out_specs=[
            pl.BlockSpec((KB, D), lambda i: (i, 0)),
            pl.BlockSpec((1, 1, KB), lambda i: (i, 0, 0)),
        ],
        out_shape=[
            jax.ShapeDtypeStruct((NFP, D), _F32),
            jax.ShapeDtypeStruct((NFP // KB, 1, KB), _F32),
        ],
    )(h, cchi, cclo, cshi, cslo)
    return re, amp


def _sc_topk_gather(amp, re):
    nvreg = NFP // 16
    mesh = plsc.VectorSubcoreMesh(core_axis_name="c", subcore_axis_name="s")

    @functools.partial(
        pl.kernel,
        mesh=mesh,
        out_type=[
            jax.ShapeDtypeStruct((TOPK,), jnp.int32),
            jax.ShapeDtypeStruct((TOPK, D), _F32),
        ],
        scratch_types=[
            pltpu.VMEM((NFP,), _F32),
            pltpu.VMEM((TOPK,), jnp.int32),
            pltpu.VMEM((TOPK, D), _F32),
            pltpu.SemaphoreType.DMA,
        ],
    )
    def sc_kernel(amp_hbm, re_hbm, idx_out, rows_out, amp_v, idx_v, rows_v, sem):
        wid = lax.axis_index("s") * 2 + lax.axis_index("c")

        @pl.when(wid == 0)
        def _():
            pltpu.sync_copy(amp_hbm, amp_v)
            lanes = lax.iota(jnp.int32, 16)

            def _perm(v, s):
                return v.at[jnp.bitwise_xor(lanes, s)].get(
                    mode="promise_in_bounds")

            def round_body(rnd, carry):
                def max_body(j, m):
                    v = amp_v[pl.ds(j * 16, 16)]
                    return jnp.maximum(m, v)

                m = lax.fori_loop(0, nvreg, max_body,
                                  jnp.full((16,), -3e38, _F32))
                for s in (8, 4, 2, 1):
                    m = jnp.maximum(m, _perm(m, s))

                def arg_body(j, bst):
                    v = amp_v[pl.ds(j * 16, 16)]
                    cand = jnp.where(v >= m, lanes + j * 16,
                                     jnp.int32(2 ** 30))
                    return jnp.minimum(bst, cand)

                best = lax.fori_loop(0, nvreg, arg_body,
                                     jnp.full((16,), 2 ** 30, jnp.int32))
                for s in (8, 4, 2, 1):
                    best = jnp.minimum(best, _perm(best, s))

                def mask_body(j, c):
                    v = amp_v[pl.ds(j * 16, 16)]
                    sel = (lanes + j * 16) == best
                    amp_v[pl.ds(j * 16, 16)] = jnp.where(sel, _F32(-3e38), v)
                    return c

                lax.fori_loop(0, nvreg, mask_body, jnp.int32(0))
                rnd16 = jnp.full((16,), 1, jnp.int32) * rnd
                for j in range(TOPK // 16):
                    cur = idx_v[pl.ds(j * 16, 16)]
                    sel = (lanes + j * 16) == rnd16
                    idx_v[pl.ds(j * 16, 16)] = jnp.where(sel, best, cur)
                return carry

            lax.fori_loop(0, TOPK, round_body, jnp.int32(0))
            pltpu.async_copy(re_hbm.at[idx_v], rows_v, sem).wait()
            pltpu.sync_copy(idx_v, idx_out)
            pltpu.sync_copy(rows_v, rows_out)

    return sc_kernel(amp, re)


def _feb_attn(r, wq, bq, wk, bk, wv, bv):
    def body(r_ref, wq_ref, bq_ref, wk_ref, bk_ref, wv_ref, bv_ref, o_ref):
        r_ = r_ref[...]
        q = _dg11(r_, wq_ref[...]) + bq_ref[...]
        k = _dg11(r_, wk_ref[...]) + bk_ref[...]
        v = _dg11(r_, wv_ref[...]) + bv_ref[...]
        s = _dg11(q, k, precision=_HI) * (1.0 / np.sqrt(D).astype(np.float32))
        s = s - s.max(axis=1, keepdims=True)
        e = jnp.exp(s)
        a = e / e.sum(axis=1, keepdims=True)
        o_ref[...] = _dg10(a, v, precision=_HI)

    return pl.pallas_call(
        body,
        out_shape=jax.ShapeDtypeStruct((TOPK, D), _F32),
    )(r, wq, bq, wk, bk, wv, bv)


def _basis_add(idx2d, ao, h):
    def body(idx_ref, ao_ref, h_ref, o_ref):
        idx = idx_ref[0, :]
        t = lax.broadcasted_iota(jnp.int32, (TSEQ, TOPK), 0)
        m = (t * idx[None, :]) % TSEQ
        ang = m.astype(_F32) * _F32(2.0 * np.pi / TSEQ)
        coef = jnp.where((idx == 0) | (idx == TSEQ // 2), 1.0, 2.0) * (
            1.0 / TSEQ)
        basis = (jnp.cos(ang) - jnp.sin(ang)) * coef[None, :].astype(_F32)
        o_ref[...] = h_ref[...] + _dg10(basis, ao_ref[...], precision=_HI)

    return pl.pallas_call(
        body,
        out_shape=jax.ShapeDtypeStruct((TSEQ, D), _F32),
    )(idx2d, ao, h)


def _mm_bias(x, w, b, nb, relu=False):
    M, K = x.shape
    NW = w.shape[0]

    def body(x_ref, w_ref, b_ref, o_ref):
        y = _dg11(x_ref[...], w_ref[...]) + b_ref[...]
        if relu:
            y = jnp.maximum(y, 0.0)
        o_ref[...] = y

    return pl.pallas_call(
        body,
        grid=(NW // nb,),
        in_specs=[
            pl.BlockSpec((M, K), lambda j: (0, 0)),
            pl.BlockSpec((nb, K), lambda j: (j, 0)),
            pl.BlockSpec((1, nb), lambda j: (0, j)),
        ],
        out_specs=pl.BlockSpec((M, nb), lambda j: (0, j)),
        out_shape=jax.ShapeDtypeStruct((M, NW), _F32),
    )(x, w, b)


def _qkv3(x, w, b3):
    def body(x_ref, w_ref, b_ref, o_ref):
        o_ref[0] = _dg11(x_ref[...], w_ref[...]) + b_ref[0]

    return pl.pallas_call(
        body,
        grid=(3 * H,),
        in_specs=[
            pl.BlockSpec((TSEQ, D), lambda j: (0, 0)),
            pl.BlockSpec((DH, D), lambda j: (j, 0)),
            pl.BlockSpec((1, 1, DH), lambda j: (j, 0, 0)),
        ],
        out_specs=pl.BlockSpec((1, TSEQ, DH), lambda j: (j, 0, 0)),
        out_shape=jax.ShapeDtypeStruct((3 * H, TSEQ, DH), _F32),
    )(x, w, b3)


def _mha_core(qkv3):
    def body(q_ref, k_ref, v_ref, o_ref):
        outs = []
        for hh in range(2):
            s = _dg11(q_ref[hh], k_ref[hh]) * (
                1.0 / np.sqrt(DH).astype(np.float32))
            s = s - s.max(axis=1, keepdims=True)
            e = jnp.exp(s)
            a = e / e.sum(axis=1, keepdims=True)
            outs.append(_dg10(a, v_ref[hh]))
        o_ref[...] = jnp.concatenate(outs, axis=1)

    return pl.pallas_call(
        body,
        grid=(H // 2, TSEQ // RB),
        in_specs=[
            pl.BlockSpec((2, RB, DH), lambda p, i: (p, i, 0)),
            pl.BlockSpec((2, TSEQ, DH), lambda p, i: (H // 2 + p, 0, 0)),
            pl.BlockSpec((2, TSEQ, DH), lambda p, i: (H + p, 0, 0)),
        ],
        out_specs=pl.BlockSpec((RB, 2 * DH), lambda p, i: (i, p)),
        out_shape=jax.ShapeDtypeStruct((TSEQ, D), _F32),
    )(qkv3, qkv3, qkv3)


def _ln(y, g, b):
    mu = y.mean(axis=1, keepdims=True)
    yc = y - mu
    var = (yc * yc).mean(axis=1, keepdims=True)
    return yc / jnp.sqrt(var + 1e-5) * g + b


def _proj_res_ln(o, w, b, res, g, bb):
    def body(o_ref, w_ref, b_ref, res_ref, g_ref, bb_ref, y_ref):
        y = _dg11(o_ref[...], w_ref[...]) + b_ref[...] + res_ref[...]
        y_ref[...] = _ln(y, g_ref[...], bb_ref[...])

    return pl.pallas_call(
        body,
        grid=(TSEQ // RB,),
        in_specs=[
            pl.BlockSpec((RB, D), lambda i: (i, 0)),
            pl.BlockSpec((D, D), lambda i: (0, 0)),
            pl.BlockSpec((1, D), lambda i: (0, 0)),
            pl.BlockSpec((RB, D), lambda i: (i, 0)),
            pl.BlockSpec((1, D), lambda i: (0, 0)),
            pl.BlockSpec((1, D), lambda i: (0, 0)),
        ],
        out_specs=pl.BlockSpec((RB, D), lambda i: (i, 0)),
        out_shape=jax.ShapeDtypeStruct((TSEQ, D), _F32),
    )(o, w, b, res, g, bb)


def _split2(a):
    hi = a.astype(jnp.bfloat16)
    lo = (a - hi.astype(_F32)).astype(jnp.bfloat16)
    return hi, lo


def _dg11_3x(x, whi, wlo):
    xhi, xlo = _split2(x)
    return (_dg11(xhi, whi) + _dg11(xlo, whi)) + _dg11(xhi, wlo)


def _ff_fused(x, w1hi, w1lo, b1, w2hi, w2lo, b2, g, bb):
    def body(x_ref, w1hi_ref, w1lo_ref, b1_ref, w2hi_ref, w2lo_ref, b2_ref,
             g_ref, bb_ref, y_ref):
        x_ = x_ref[...]
        a = jnp.maximum(
            _dg11_3x(x_, w1hi_ref[...], w1lo_ref[...]) + b1_ref[...], 0.0)
        y = _dg11_3x(a, w2hi_ref[...], w2lo_ref[...]) + b2_ref[...] + x_
        y_ref[...] = _ln(y, g_ref[...], bb_ref[...])

    return pl.pallas_call(
        body,
        grid=(TSEQ // RB,),
        in_specs=[
            pl.BlockSpec((RB, D), lambda i: (i, 0)),
            pl.BlockSpec((DFF, D), lambda i: (0, 0)),
            pl.BlockSpec((DFF, D), lambda i: (0, 0)),
            pl.BlockSpec((1, DFF), lambda i: (0, 0)),
            pl.BlockSpec((D, DFF), lambda i: (0, 0)),
            pl.BlockSpec((D, DFF), lambda i: (0, 0)),
            pl.BlockSpec((1, D), lambda i: (0, 0)),
            pl.BlockSpec((1, D), lambda i: (0, 0)),
            pl.BlockSpec((1, D), lambda i: (0, 0)),
        ],
        out_specs=pl.BlockSpec((RB, D), lambda i: (i, 0)),
        out_shape=jax.ShapeDtypeStruct((TSEQ, D), _F32),
    )(x, w1hi, w1lo, b1, w2hi, w2lo, b2, g, bb)


def _pool_fc(h, w, b):
    def body(h_ref, w_ref, b_ref, o_ref):
        pooled = h_ref[...].mean(axis=0, keepdims=True)
        o_ref[...] = _dg11(pooled, w_ref[...], precision=_HI) + b_ref[...]

    nout = w.shape[0]
    return pl.pallas_call(
        body,
        out_shape=jax.ShapeDtypeStruct((1, nout), _F32),
    )(h, w, b)


def kernel(x, params):
    cchi = jnp.asarray(_CCOS_HI)
    cclo = jnp.asarray(_CCOS_LO)
    cshi = jnp.asarray(_CSIN_HI)
    cslo = jnp.asarray(_CSIN_LO)
    pe = jnp.asarray(_PE_NP)

    x2 = x[0]
    xp = jnp.pad(x2, ((2 * PADC, 2 * PADC), (0, 0)))
    h = _embed(xp, params['in_w'], params['in_b'][None], pe)

    for lp in params['layers']:
        re, amp = _spectrum(h, cchi, cclo, cshi, cslo)
        idx, r = _sc_topk_gather(amp.reshape(NFP), re)
        ao = _feb_attn(r, lp['wq'], lp['bq'][None], lp['wk'], lp['bk'][None],
                       lp['wv'], lp['bv'][None])
        h = _basis_add(idx.reshape(1, TOPK), ao, h)

        qkv3 = _qkv3(h, lp['win'], lp['bin'].reshape(3 * H, 1, DH))
        o = _mha_core(qkv3)
        h = _proj_res_ln(o, lp['wout'], lp['bout'][None], h,
                         lp['n1g'][None], lp['n1b'][None])
        w1hi, w1lo = _split2(lp['w1'])
        w2hi, w2lo = _split2(lp['w2'])
        h = _ff_fused(h, w1hi, w1lo, lp['b1'][None], w2hi, w2lo,
                      lp['b2'][None], lp['n2g'][None], lp['n2b'][None])

    out = _pool_fc(h, params['fc_w'], params['fc_b'][None])
    return out.reshape(1, 96, 8)


# speedup vs baseline: 1.2520x; 1.2520x over previous
import functools

import numpy as np
import jax
import jax.numpy as jnp
from jax import lax
from jax.experimental import pallas as pl
from jax.experimental.pallas import tpu as pltpu
from jax.experimental.pallas import tpu_sc as plsc

TIN = 2048
FIN = 256
TSEQ = 2072
NF = TSEQ // 2 + 1
NFP = 1040
D = 768
H = 12
DH = D // H
DFF = 3072
TOPK = 32
KDEC = 25
PADC = (KDEC - 1) // 2
RB = 296
KB = 208

_F32 = jnp.float32

def _dft_consts():
    t = np.arange(TSEQ, dtype=np.int64)
    k = np.arange(NFP, dtype=np.int64)
    m = (k[:, None] * t[None, :]) % TSEQ
    ang = (2.0 * np.pi / TSEQ) * m
    cc = np.cos(ang)
    cs = -np.sin(ang)
    cc[NF:] = 0.0
    cs[NF:] = 0.0
    return cc.astype(np.float32), cs.astype(np.float32)


def _pe_const():
    pos = np.arange(TSEQ, dtype=np.float64)[:, None]
    div = np.exp(np.arange(0, D, 2, dtype=np.float64) * (-np.log(10000.0) / D))
    pe = np.zeros((TSEQ, D), np.float64)
    pe[:, 0::2] = np.sin(pos * div)
    pe[:, 1::2] = np.cos(pos * div)
    return pe.astype(np.float32)


_CCOS_NP, _CSIN_NP = _dft_consts()
_PE_NP = _pe_const()


def _split_bf16(a):
    hi = a.astype(np.float32).astype(jnp.bfloat16)
    lo = (a - np.asarray(hi, np.float32)).astype(jnp.bfloat16)
    return np.asarray(hi), np.asarray(lo)


_CCOS_HI, _CCOS_LO = _split_bf16(_CCOS_NP)
_CSIN_HI, _CSIN_LO = _split_bf16(_CSIN_NP)

_HI = lax.Precision.HIGHEST


def _dg11(a, b, precision=None):
    return lax.dot_general(a, b, (((1,), (1,)), ((), ())),
                           precision=precision, preferred_element_type=_F32)


def _dg10(a, b, precision=None):
    return lax.dot_general(a, b, (((1,), (0,)), ((), ())),
                           precision=precision, preferred_element_type=_F32)


def _embed(xp, w, b, pe):
    def body(xp_ref, w_ref, b_ref, pe_ref, o_ref):
        acc = xp_ref[0:TSEQ, :]
        for j in range(1, KDEC):
            acc = acc + xp_ref[j:j + TSEQ, :]
        seasonal = xp_ref[PADC:PADC + TSEQ, :] - acc * (1.0 / KDEC)
        h = _dg11(seasonal, w_ref[...])
        o_ref[...] = h + b_ref[...] + pe_ref[...]

    return pl.pallas_call(
        body,
        out_shape=jax.ShapeDtypeStruct((TSEQ, D), _F32),
    )(xp, w, b, pe)


def _spectrum(h, cchi, cclo, cshi, cslo):
    def body(h_ref, cchi_ref, cclo_ref, cshi_ref, cslo_ref, re_ref, amp_ref):
        h_ = h_ref[...]
        h_hi = h_.astype(jnp.bfloat16)
        h_lo = (h_ - h_hi.astype(_F32)).astype(jnp.bfloat16)
        re = (_dg10(cchi_ref[...], h_hi) + _dg10(cchi_ref[...], h_lo)
              + _dg10(cclo_ref[...], h_hi))
        im = (_dg10(cshi_ref[...], h_hi) + _dg10(cshi_ref[...], h_lo)
              + _dg10(cslo_ref[...], h_hi))
        re_ref[...] = re
        amp = jnp.sqrt(re * re + im * im).mean(axis=1)
        i = pl.program_id(0)
        row = i * KB + lax.broadcasted_iota(jnp.int32, (1, 1, KB), 2)
        amp_ref[...] = jnp.where(row < NF, amp[None, None, :], -1.0)

    re, amp = pl.pallas_call(
        body,
        grid=(NFP // KB,),
        in_specs=[
            pl.BlockSpec((TSEQ, D), lambda i: (0, 0)),
            pl.BlockSpec((KB, TSEQ), lambda i: (i, 0)),
            pl.BlockSpec((KB, TSEQ), lambda i: (i, 0)),
            pl.BlockSpec((KB, TSEQ), lambda i: (i, 0)),
            pl.BlockSpec((KB, TSEQ), lambda i: (i, 0)),
        ],
        out_specs=[
            pl.BlockSpec((KB, D), lambda i: (i, 0)),
            pl.BlockSpec((1, 1, KB), lambda i: (i, 0, 0)),
        ],
        out_shape=[
            jax.ShapeDtypeStruct((NFP, D), _F32),
            jax.ShapeDtypeStruct((NFP // KB, 1, KB), _F32),
        ],
    )(h, cchi, cclo, cshi, cslo)
    return re, amp


def _sc_topk_gather(amp, re):
    nvreg = NFP // 16
    mesh = plsc.VectorSubcoreMesh(core_axis_name="c", subcore_axis_name="s")

    @functools.partial(
        pl.kernel,
        mesh=mesh,
        out_type=[
            jax.ShapeDtypeStruct((TOPK,), jnp.int32),
            jax.ShapeDtypeStruct((TOPK, D), _F32),
        ],
        scratch_types=[
            pltpu.VMEM((NFP,), _F32),
            pltpu.VMEM((TOPK,), jnp.int32),
            pltpu.VMEM((TOPK, D), _F32),
            pltpu.SemaphoreType.DMA,
        ],
    )
    def sc_kernel(amp_hbm, re_hbm, idx_out, rows_out, amp_v, idx_v, rows_v, sem):
        wid = lax.axis_index("s") * 2 + lax.axis_index("c")

        @pl.when(wid == 0)
        def _():
            pltpu.sync_copy(amp_hbm, amp_v)
            lanes = lax.iota(jnp.int32, 16)

            def _perm(v, s):
                return v.at[jnp.bitwise_xor(lanes, s)].get(
                    mode="promise_in_bounds")

            def round_body(rnd, best_prev):
                m16 = jnp.full((16,), -3e38, _F32)
                a16 = jnp.full((16,), 2 ** 30, jnp.int32)
                for j in range(nvreg):
                    v = amp_v[pl.ds(j * 16, 16)]
                    idxv = lanes + j * 16
                    v = jnp.where(idxv == best_prev, _F32(-3e38), v)
                    amp_v[pl.ds(j * 16, 16)] = v
                    take = v > m16
                    a16 = jnp.where(take, idxv, a16)
                    m16 = jnp.maximum(m16, v)
                for s in (8, 4, 2, 1):
                    mp = _perm(m16, s)
                    ap = _perm(a16, s)
                    take = (mp > m16) | ((mp == m16) & (ap < a16))
                    a16 = jnp.where(take, ap, a16)
                    m16 = jnp.where(take, mp, m16)
                rnd16 = jnp.full((16,), 1, jnp.int32) * rnd
                for j in range(TOPK // 16):
                    cur = idx_v[pl.ds(j * 16, 16)]
                    sel = (lanes + j * 16) == rnd16
                    idx_v[pl.ds(j * 16, 16)] = jnp.where(sel, a16, cur)
                return a16

            lax.fori_loop(0, TOPK, round_body,
                          jnp.full((16,), -1, jnp.int32))
            pltpu.async_copy(re_hbm.at[idx_v], rows_v, sem).wait()
            pltpu.sync_copy(idx_v, idx_out)
            pltpu.sync_copy(rows_v, rows_out)

    return sc_kernel(amp, re)


def _feb_apply(idx2d, r, wq, bq, wk, bk, wv, bv, h):
    def body(idx_ref, r_ref, wq_ref, bq_ref, wk_ref, bk_ref, wv_ref, bv_ref,
             h_ref, o_ref):
        r_ = r_ref[...]
        q = _dg11(r_, wq_ref[...]) + bq_ref[...]
        k = _dg11(r_, wk_ref[...]) + bk_ref[...]
        v = _dg11(r_, wv_ref[...]) + bv_ref[...]
        s = _dg11(q, k, precision=_HI) * (1.0 / np.sqrt(D).astype(np.float32))
        s = s - s.max(axis=1, keepdims=True)
        e = jnp.exp(s)
        a = e / e.sum(axis=1, keepdims=True)
        ao = _dg10(a, v, precision=_HI)

        idx = idx_ref[0, :]
        t = lax.broadcasted_iota(jnp.int32, (TSEQ, TOPK), 0)
        m = (t * idx[None, :]) % TSEQ
        ang = m.astype(_F32) * _F32(2.0 * np.pi / TSEQ)
        coef = jnp.where((idx == 0) | (idx == TSEQ // 2), 1.0, 2.0) * (
            1.0 / TSEQ)
        basis = (jnp.cos(ang) - jnp.sin(ang)) * coef[None, :].astype(_F32)
        o_ref[...] = h_ref[...] + _dg10(basis, ao, precision=_HI)

    return pl.pallas_call(
        body,
        out_shape=jax.ShapeDtypeStruct((TSEQ, D), _F32),
    )(idx2d, r, wq, bq, wk, bk, wv, bv, h)


def _mm_bias(x, w, b, nb, relu=False):
    M, K = x.shape
    NW = w.shape[0]

    def body(x_ref, w_ref, b_ref, o_ref):
        y = _dg11(x_ref[...], w_ref[...]) + b_ref[...]
        if relu:
            y = jnp.maximum(y, 0.0)
        o_ref[...] = y

    return pl.pallas_call(
        body,
        grid=(NW // nb,),
        in_specs=[
            pl.BlockSpec((M, K), lambda j: (0, 0)),
            pl.BlockSpec((nb, K), lambda j: (j, 0)),
            pl.BlockSpec((1, nb), lambda j: (0, j)),
        ],
        out_specs=pl.BlockSpec((M, nb), lambda j: (0, j)),
        out_shape=jax.ShapeDtypeStruct((M, NW), _F32),
    )(x, w, b)


def _qkv3(x, w, b3):
    def body(x_ref, w_ref, b_ref, o_ref):
        o_ref[0] = _dg11(x_ref[...], w_ref[...]) + b_ref[0]

    return pl.pallas_call(
        body,
        grid=(3 * H,),
        in_specs=[
            pl.BlockSpec((TSEQ, D), lambda j: (0, 0)),
            pl.BlockSpec((DH, D), lambda j: (j, 0)),
            pl.BlockSpec((1, 1, DH), lambda j: (j, 0, 0)),
        ],
        out_specs=pl.BlockSpec((1, TSEQ, DH), lambda j: (j, 0, 0)),
        out_shape=jax.ShapeDtypeStruct((3 * H, TSEQ, DH), _F32),
    )(x, w, b3)


def _mha_core(qkv3):
    def body(q_ref, k_ref, v_ref, o_ref):
        outs = []
        for hh in range(2):
            s = _dg11(q_ref[hh], k_ref[hh]) * (
                1.0 / np.sqrt(DH).astype(np.float32))
            s = s - s.max(axis=1, keepdims=True)
            e = jnp.exp(s)
            a = e / e.sum(axis=1, keepdims=True)
            outs.append(_dg10(a, v_ref[hh]))
        o_ref[...] = jnp.concatenate(outs, axis=1)

    return pl.pallas_call(
        body,
        grid=(H // 2, TSEQ // RB),
        in_specs=[
            pl.BlockSpec((2, RB, DH), lambda p, i: (p, i, 0)),
            pl.BlockSpec((2, TSEQ, DH), lambda p, i: (H // 2 + p, 0, 0)),
            pl.BlockSpec((2, TSEQ, DH), lambda p, i: (H + p, 0, 0)),
        ],
        out_specs=pl.BlockSpec((RB, 2 * DH), lambda p, i: (i, p)),
        out_shape=jax.ShapeDtypeStruct((TSEQ, D), _F32),
    )(qkv3, qkv3, qkv3)


def _ln(y, g, b):
    mu = y.mean(axis=1, keepdims=True)
    yc = y - mu
    var = (yc * yc).mean(axis=1, keepdims=True)
    return yc / jnp.sqrt(var + 1e-5) * g + b


def _encoder_tail(o, wout, bout, res, g1, bn1, w1, b1, w2, b2, g2, bn2):
    def body(o_ref, wout_ref, bout_ref, res_ref, g1_ref, bn1_ref,
             w1_ref, b1_ref, w2_ref, b2_ref, g2_ref, bn2_ref, y_ref):
        x1 = _ln(_dg11(o_ref[...], wout_ref[...]) + bout_ref[...]
                 + res_ref[...], g1_ref[...], bn1_ref[...])
        a = jnp.maximum(_dg11(x1, w1_ref[...]) + b1_ref[...], 0.0)
        y = _dg11(a, w2_ref[...]) + b2_ref[...] + x1
        y_ref[...] = _ln(y, g2_ref[...], bn2_ref[...])

    vec = pl.BlockSpec((1, D), lambda i: (0, 0))
    return pl.pallas_call(
        body,
        grid=(TSEQ // RB,),
        in_specs=[
            pl.BlockSpec((RB, D), lambda i: (i, 0)),
            pl.BlockSpec((D, D), lambda i: (0, 0)),
            vec,
            pl.BlockSpec((RB, D), lambda i: (i, 0)),
            vec, vec,
            pl.BlockSpec((DFF, D), lambda i: (0, 0)),
            pl.BlockSpec((1, DFF), lambda i: (0, 0)),
            pl.BlockSpec((D, DFF), lambda i: (0, 0)),
            vec, vec, vec,
        ],
        out_specs=pl.BlockSpec((RB, D), lambda i: (i, 0)),
        out_shape=jax.ShapeDtypeStruct((TSEQ, D), _F32),
    )(o, wout, bout, res, g1, bn1, w1, b1, w2, b2, g2, bn2)


def _pool_fc(h, w, b):
    def body(h_ref, w_ref, b_ref, o_ref):
        pooled = h_ref[...].mean(axis=0, keepdims=True)
        o_ref[...] = _dg11(pooled, w_ref[...], precision=_HI) + b_ref[...]

    nout = w.shape[0]
    return pl.pallas_call(
        body,
        out_shape=jax.ShapeDtypeStruct((1, nout), _F32),
    )(h, w, b)


def kernel(x, params):
    cchi = jnp.asarray(_CCOS_HI)
    cclo = jnp.asarray(_CCOS_LO)
    cshi = jnp.asarray(_CSIN_HI)
    cslo = jnp.asarray(_CSIN_LO)
    pe = jnp.asarray(_PE_NP)

    x2 = x[0]
    xp = jnp.pad(x2, ((2 * PADC, 2 * PADC), (0, 0)))
    h = _embed(xp, params['in_w'], params['in_b'][None], pe)

    for lp in params['layers']:
        re, amp = _spectrum(h, cchi, cclo, cshi, cslo)
        idx, r = _sc_topk_gather(amp.reshape(NFP), re)
        h = _feb_apply(idx.reshape(1, TOPK), r, lp['wq'], lp['bq'][None],
                       lp['wk'], lp['bk'][None], lp['wv'], lp['bv'][None], h)

        qkv3 = _qkv3(h, lp['win'], lp['bin'].reshape(3 * H, 1, DH))
        o = _mha_core(qkv3)
        h = _encoder_tail(o, lp['wout'], lp['bout'][None], h,
                          lp['n1g'][None], lp['n1b'][None],
                          lp['w1'], lp['b1'][None], lp['w2'], lp['b2'][None],
                          lp['n2g'][None], lp['n2b'][None])

    out = _pool_fc(h, params['fc_w'], params['fc_b'][None])
    return out.reshape(1, 96, 8)


# speedup vs baseline: 1.5446x; 1.2337x over previous
import functools

import numpy as np
import jax
import jax.numpy as jnp
from jax import lax
from jax.experimental import pallas as pl
from jax.experimental.pallas import tpu as pltpu
from jax.experimental.pallas import tpu_sc as plsc

TIN = 2048
FIN = 256
TSEQ = 2072
NF = TSEQ // 2 + 1
NFP = 1040
D = 768
H = 12
DH = D // H
DFF = 3072
TOPK = 32
KDEC = 25
PADC = (KDEC - 1) // 2
RB = 296
KB = 208

_F32 = jnp.float32

def _dft_consts():
    t = np.arange(TSEQ, dtype=np.int64)
    k = np.arange(NFP, dtype=np.int64)
    m = (k[:, None] * t[None, :]) % TSEQ
    ang = (2.0 * np.pi / TSEQ) * m
    cc = np.cos(ang)
    cs = -np.sin(ang)
    cc[NF:] = 0.0
    cs[NF:] = 0.0
    return cc.astype(np.float32), cs.astype(np.float32)


def _pe_const():
    pos = np.arange(TSEQ, dtype=np.float64)[:, None]
    div = np.exp(np.arange(0, D, 2, dtype=np.float64) * (-np.log(10000.0) / D))
    pe = np.zeros((TSEQ, D), np.float64)
    pe[:, 0::2] = np.sin(pos * div)
    pe[:, 1::2] = np.cos(pos * div)
    return pe.astype(np.float32)


_CCOS_NP, _CSIN_NP = _dft_consts()
_PE_NP = _pe_const()


def _split_bf16(a):
    hi = a.astype(np.float32).astype(jnp.bfloat16)
    lo = (a - np.asarray(hi, np.float32)).astype(jnp.bfloat16)
    return np.asarray(hi), np.asarray(lo)


_CCOS_HI, _CCOS_LO = _split_bf16(_CCOS_NP)
_CSIN_HI, _CSIN_LO = _split_bf16(_CSIN_NP)

_HI = lax.Precision.HIGHEST


def _dg11(a, b, precision=None):
    return lax.dot_general(a, b, (((1,), (1,)), ((), ())),
                           precision=precision, preferred_element_type=_F32)


def _dg10(a, b, precision=None):
    return lax.dot_general(a, b, (((1,), (0,)), ((), ())),
                           precision=precision, preferred_element_type=_F32)


def _embed(xp, w, b, pe):
    def body(xp_ref, w_ref, b_ref, pe_ref, o_ref):
        acc = xp_ref[0:TSEQ, :]
        for j in range(1, KDEC):
            acc = acc + xp_ref[j:j + TSEQ, :]
        seasonal = xp_ref[PADC:PADC + TSEQ, :] - acc * (1.0 / KDEC)
        h = _dg11(seasonal, w_ref[...])
        o_ref[...] = h + b_ref[...] + pe_ref[...]

    return pl.pallas_call(
        body,
        out_shape=jax.ShapeDtypeStruct((TSEQ, D), _F32),
    )(xp, w, b, pe)


def _spectrum(h, cchi, cclo, cshi, cslo):
    def body(h_ref, cchi_ref, cclo_ref, cshi_ref, cslo_ref, re_ref, amp_ref):
        h_ = h_ref[...]
        h_hi = h_.astype(jnp.bfloat16)
        h_lo = (h_ - h_hi.astype(_F32)).astype(jnp.bfloat16)
        re = (_dg10(cchi_ref[...], h_hi) + _dg10(cchi_ref[...], h_lo)
              + _dg10(cclo_ref[...], h_hi))
        im = (_dg10(cshi_ref[...], h_hi) + _dg10(cshi_ref[...], h_lo)
              + _dg10(cslo_ref[...], h_hi))
        re_ref[...] = re
        amp = jnp.sqrt(re * re + im * im).mean(axis=1)
        i = pl.program_id(0)
        row = i * KB + lax.broadcasted_iota(jnp.int32, (1, 1, KB), 2)
        amp_ref[...] = jnp.where(row < NF, amp[None, None, :], -1.0)

    re, amp = pl.pallas_call(
        body,
        grid=(NFP // KB,),
        in_specs=[
            pl.BlockSpec((TSEQ, D), lambda i: (0, 0)),
            pl.BlockSpec((KB, TSEQ), lambda i: (i, 0)),
            pl.BlockSpec((KB, TSEQ), lambda i: (i, 0)),
            pl.BlockSpec((KB, TSEQ), lambda i: (i, 0)),
            pl.BlockSpec((KB, TSEQ), lambda i: (i, 0)),
        ],
        out_specs=[
            pl.BlockSpec((KB, D), lambda i: (i, 0)),
            pl.BlockSpec((1, 1, KB), lambda i: (i, 0, 0)),
        ],
        out_shape=[
            jax.ShapeDtypeStruct((NFP, D), _F32),
            jax.ShapeDtypeStruct((NFP // KB, 1, KB), _F32),
        ],
    )(h, cchi, cclo, cshi, cslo)
    return re, amp


def _sc_topk_gather(amp, re):
    nvreg = NFP // 16
    mesh = plsc.VectorSubcoreMesh(core_axis_name="c", subcore_axis_name="s")

    @functools.partial(
        pl.kernel,
        mesh=mesh,
        out_type=[
            jax.ShapeDtypeStruct((TOPK,), jnp.int32),
            jax.ShapeDtypeStruct((TOPK, D), _F32),
        ],
        scratch_types=[
            pltpu.VMEM((NFP,), _F32),
            pltpu.VMEM((TOPK,), jnp.int32),
            pltpu.VMEM((TOPK, D), _F32),
            pltpu.SemaphoreType.DMA,
        ],
    )
    def sc_kernel(amp_hbm, re_hbm, idx_out, rows_out, amp_v, idx_v, rows_v, sem):
        wid = lax.axis_index("s") * 2 + lax.axis_index("c")

        @pl.when(wid == 0)
        def _():
            pltpu.sync_copy(amp_hbm, amp_v)
            lanes = lax.iota(jnp.int32, 16)

            def _perm(v, s):
                return v.at[jnp.bitwise_xor(lanes, s)].get(
                    mode="promise_in_bounds")

            def round_body(rnd, best_prev):
                m16 = jnp.full((16,), -3e38, _F32)
                a16 = jnp.full((16,), 2 ** 30, jnp.int32)
                for j in range(nvreg):
                    v = amp_v[pl.ds(j * 16, 16)]
                    idxv = lanes + j * 16
                    v = jnp.where(idxv == best_prev, _F32(-3e38), v)
                    amp_v[pl.ds(j * 16, 16)] = v
                    take = v > m16
                    a16 = jnp.where(take, idxv, a16)
                    m16 = jnp.maximum(m16, v)
                for s in (8, 4, 2, 1):
                    mp = _perm(m16, s)
                    ap = _perm(a16, s)
                    take = (mp > m16) | ((mp == m16) & (ap < a16))
                    a16 = jnp.where(take, ap, a16)
                    m16 = jnp.where(take, mp, m16)
                rnd16 = jnp.full((16,), 1, jnp.int32) * rnd
                for j in range(TOPK // 16):
                    cur = idx_v[pl.ds(j * 16, 16)]
                    sel = (lanes + j * 16) == rnd16
                    idx_v[pl.ds(j * 16, 16)] = jnp.where(sel, a16, cur)
                return a16

            lax.fori_loop(0, TOPK, round_body,
                          jnp.full((16,), -1, jnp.int32))
            pltpu.async_copy(re_hbm.at[idx_v], rows_v, sem).wait()
            pltpu.sync_copy(idx_v, idx_out)
            pltpu.sync_copy(rows_v, rows_out)

    return sc_kernel(amp, re)


def _feb_apply(idx2d, r, wq, bq, wk, bk, wv, bv, h):
    def body(idx_ref, r_ref, wq_ref, bq_ref, wk_ref, bk_ref, wv_ref, bv_ref,
             h_ref, o_ref):
        r_ = r_ref[...]
        q = _dg11(r_, wq_ref[...]) + bq_ref[...]
        k = _dg11(r_, wk_ref[...]) + bk_ref[...]
        v = _dg11(r_, wv_ref[...]) + bv_ref[...]
        s = _dg11(q, k, precision=_HI) * (1.0 / np.sqrt(D).astype(np.float32))
        s = s - s.max(axis=1, keepdims=True)
        e = jnp.exp(s)
        a = e / e.sum(axis=1, keepdims=True)
        ao = _dg10(a, v, precision=_HI)

        idx = idx_ref[0, :]
        t = lax.broadcasted_iota(jnp.int32, (TSEQ, TOPK), 0)
        m = (t * idx[None, :]) % TSEQ
        ang = m.astype(_F32) * _F32(2.0 * np.pi / TSEQ)
        coef = jnp.where((idx == 0) | (idx == TSEQ // 2), 1.0, 2.0) * (
            1.0 / TSEQ)
        basis = (jnp.cos(ang) - jnp.sin(ang)) * coef[None, :].astype(_F32)
        o_ref[...] = h_ref[...] + _dg10(basis, ao, precision=_HI)

    return pl.pallas_call(
        body,
        out_shape=jax.ShapeDtypeStruct((TSEQ, D), _F32),
    )(idx2d, r, wq, bq, wk, bk, wv, bv, h)


def _mm_bias(x, w, b, nb, relu=False):
    M, K = x.shape
    NW = w.shape[0]

    def body(x_ref, w_ref, b_ref, o_ref):
        y = _dg11(x_ref[...], w_ref[...]) + b_ref[...]
        if relu:
            y = jnp.maximum(y, 0.0)
        o_ref[...] = y

    return pl.pallas_call(
        body,
        grid=(NW // nb,),
        in_specs=[
            pl.BlockSpec((M, K), lambda j: (0, 0)),
            pl.BlockSpec((nb, K), lambda j: (j, 0)),
            pl.BlockSpec((1, nb), lambda j: (0, j)),
        ],
        out_specs=pl.BlockSpec((M, nb), lambda j: (0, j)),
        out_shape=jax.ShapeDtypeStruct((M, NW), _F32),
    )(x, w, b)


def _qkv3(x, w, b3):
    def body(x_ref, w_ref, b_ref, o_ref):
        o_ref[0] = _dg11(x_ref[...], w_ref[...]) + b_ref[0]

    return pl.pallas_call(
        body,
        grid=(3 * H,),
        in_specs=[
            pl.BlockSpec((TSEQ, D), lambda j: (0, 0)),
            pl.BlockSpec((DH, D), lambda j: (j, 0)),
            pl.BlockSpec((1, 1, DH), lambda j: (j, 0, 0)),
        ],
        out_specs=pl.BlockSpec((1, TSEQ, DH), lambda j: (j, 0, 0)),
        out_shape=jax.ShapeDtypeStruct((3 * H, TSEQ, DH), _F32),
    )(x, w, b3)


def _mha_core(qkv3):
    def body(q_ref, k_ref, v_ref, o_ref):
        outs = []
        for hh in range(2):
            s = _dg11(q_ref[hh], k_ref[hh]) * (
                1.0 / np.sqrt(DH).astype(np.float32))
            e = jnp.exp(s)
            recip = 1.0 / e.sum(axis=1, keepdims=True)
            outs.append(_dg10(e, v_ref[hh]) * recip)
        o_ref[...] = jnp.concatenate(outs, axis=1)

    return pl.pallas_call(
        body,
        grid=(H // 2, TSEQ // RB),
        in_specs=[
            pl.BlockSpec((2, RB, DH), lambda p, i: (p, i, 0)),
            pl.BlockSpec((2, TSEQ, DH), lambda p, i: (H // 2 + p, 0, 0)),
            pl.BlockSpec((2, TSEQ, DH), lambda p, i: (H + p, 0, 0)),
        ],
        out_specs=pl.BlockSpec((RB, 2 * DH), lambda p, i: (i, p)),
        out_shape=jax.ShapeDtypeStruct((TSEQ, D), _F32),
    )(qkv3, qkv3, qkv3)


def _ln(y, g, b):
    mu = y.mean(axis=1, keepdims=True)
    yc = y - mu
    var = (yc * yc).mean(axis=1, keepdims=True)
    return yc / jnp.sqrt(var + 1e-5) * g + b


def _encoder_tail(o, wout, bout, res, g1, bn1, w1, b1, w2, b2, g2, bn2):
    def body(o_ref, wout_ref, bout_ref, res_ref, g1_ref, bn1_ref,
             w1_ref, b1_ref, w2_ref, b2_ref, g2_ref, bn2_ref, y_ref):
        x1 = _ln(_dg11(o_ref[...], wout_ref[...]) + bout_ref[...]
                 + res_ref[...], g1_ref[...], bn1_ref[...])
        a = jnp.maximum(_dg11(x1, w1_ref[...]) + b1_ref[...], 0.0)
        y = _dg11(a, w2_ref[...]) + b2_ref[...] + x1
        y_ref[...] = _ln(y, g2_ref[...], bn2_ref[...])

    vec = pl.BlockSpec((1, D), lambda i: (0, 0))
    return pl.pallas_call(
        body,
        grid=(TSEQ // RB,),
        in_specs=[
            pl.BlockSpec((RB, D), lambda i: (i, 0)),
            pl.BlockSpec((D, D), lambda i: (0, 0)),
            vec,
            pl.BlockSpec((RB, D), lambda i: (i, 0)),
            vec, vec,
            pl.BlockSpec((DFF, D), lambda i: (0, 0)),
            pl.BlockSpec((1, DFF), lambda i: (0, 0)),
            pl.BlockSpec((D, DFF), lambda i: (0, 0)),
            vec, vec, vec,
        ],
        out_specs=pl.BlockSpec((RB, D), lambda i: (i, 0)),
        out_shape=jax.ShapeDtypeStruct((TSEQ, D), _F32),
    )(o, wout, bout, res, g1, bn1, w1, b1, w2, b2, g2, bn2)


def _pool_fc(h, w, b):
    def body(h_ref, w_ref, b_ref, o_ref):
        pooled = h_ref[...].mean(axis=0, keepdims=True)
        o_ref[...] = _dg11(pooled, w_ref[...], precision=_HI) + b_ref[...]

    nout = w.shape[0]
    return pl.pallas_call(
        body,
        out_shape=jax.ShapeDtypeStruct((1, nout), _F32),
    )(h, w, b)


def kernel(x, params):
    cchi = jnp.asarray(_CCOS_HI)
    cclo = jnp.asarray(_CCOS_LO)
    cshi = jnp.asarray(_CSIN_HI)
    cslo = jnp.asarray(_CSIN_LO)
    pe = jnp.asarray(_PE_NP)

    x2 = x[0]
    xp = jnp.pad(x2, ((2 * PADC, 2 * PADC), (0, 0)))
    h = _embed(xp, params['in_w'], params['in_b'][None], pe)

    for lp in params['layers']:
        re, amp = _spectrum(h, cchi, cclo, cshi, cslo)
        idx, r = _sc_topk_gather(amp.reshape(NFP), re)
        h = _feb_apply(idx.reshape(1, TOPK), r, lp['wq'], lp['bq'][None],
                       lp['wk'], lp['bk'][None], lp['wv'], lp['bv'][None], h)

        qkv3 = _qkv3(h, lp['win'], lp['bin'].reshape(3 * H, 1, DH))
        o = _mha_core(qkv3)
        h = _encoder_tail(o, lp['wout'], lp['bout'][None], h,
                          lp['n1g'][None], lp['n1b'][None],
                          lp['w1'], lp['b1'][None], lp['w2'], lp['b2'][None],
                          lp['n2g'][None], lp['n2b'][None])

    out = _pool_fc(h, params['fc_w'], params['fc_b'][None])
    return out.reshape(1, 96, 8)


# speedup vs baseline: 1.6417x; 1.0629x over previous
import functools

import numpy as np
import jax
import jax.numpy as jnp
from jax import lax
from jax.experimental import pallas as pl
from jax.experimental.pallas import tpu as pltpu
from jax.experimental.pallas import tpu_sc as plsc

TIN = 2048
FIN = 256
TSEQ = 2072
NF = TSEQ // 2 + 1
NFP = 1040
D = 768
H = 12
DH = D // H
DFF = 3072
TOPK = 32
KDEC = 25
PADC = (KDEC - 1) // 2
RB = 296
KB = 208

_F32 = jnp.float32

def _dft_consts():
    t = np.arange(TSEQ, dtype=np.int64)
    k = np.arange(NFP, dtype=np.int64)
    m = (k[:, None] * t[None, :]) % TSEQ
    ang = (2.0 * np.pi / TSEQ) * m
    cc = np.cos(ang)
    cs = -np.sin(ang)
    cc[NF:] = 0.0
    cs[NF:] = 0.0
    return cc.astype(np.float32), cs.astype(np.float32)


def _pe_const():
    pos = np.arange(TSEQ, dtype=np.float64)[:, None]
    div = np.exp(np.arange(0, D, 2, dtype=np.float64) * (-np.log(10000.0) / D))
    pe = np.zeros((TSEQ, D), np.float64)
    pe[:, 0::2] = np.sin(pos * div)
    pe[:, 1::2] = np.cos(pos * div)
    return pe.astype(np.float32)


_CCOS_NP, _CSIN_NP = _dft_consts()
_PE_NP = _pe_const()
TSEQP = 2176
_BASIS_NP = np.zeros((NFP, TSEQP), np.float32)
_BASIS_NP[:, :TSEQ] = _CCOS_NP + _CSIN_NP


def _split_bf16(a):
    hi = a.astype(np.float32).astype(jnp.bfloat16)
    lo = (a - np.asarray(hi, np.float32)).astype(jnp.bfloat16)
    return np.asarray(hi), np.asarray(lo)


_CCOS_HI, _CCOS_LO = _split_bf16(_CCOS_NP)
_CSIN_HI, _CSIN_LO = _split_bf16(_CSIN_NP)

_HI = lax.Precision.HIGHEST


def _dg11(a, b, precision=None):
    return lax.dot_general(a, b, (((1,), (1,)), ((), ())),
                           precision=precision, preferred_element_type=_F32)


def _dg10(a, b, precision=None):
    return lax.dot_general(a, b, (((1,), (0,)), ((), ())),
                           precision=precision, preferred_element_type=_F32)


def _embed(xp, w, b, pe):
    def body(xp_ref, w_ref, b_ref, pe_ref, o_ref):
        acc = xp_ref[0:TSEQ, :]
        for j in range(1, KDEC):
            acc = acc + xp_ref[j:j + TSEQ, :]
        seasonal = xp_ref[PADC:PADC + TSEQ, :] - acc * (1.0 / KDEC)
        h = _dg11(seasonal, w_ref[...])
        o_ref[...] = h + b_ref[...] + pe_ref[...]

    return pl.pallas_call(
        body,
        out_shape=jax.ShapeDtypeStruct((TSEQ, D), _F32),
    )(xp, w, b, pe)


def _spectrum(h, cchi, cclo, cshi, cslo):
    def body(h_ref, cchi_ref, cclo_ref, cshi_ref, cslo_ref, re_ref, amp_ref):
        h_ = h_ref[...]
        h_hi = h_.astype(jnp.bfloat16)
        h_lo = (h_ - h_hi.astype(_F32)).astype(jnp.bfloat16)
        re = (_dg10(cchi_ref[...], h_hi) + _dg10(cchi_ref[...], h_lo)
              + _dg10(cclo_ref[...], h_hi))
        im = (_dg10(cshi_ref[...], h_hi) + _dg10(cshi_ref[...], h_lo)
              + _dg10(cslo_ref[...], h_hi))
        re_ref[...] = re
        amp = jnp.sqrt(re * re + im * im).mean(axis=1)
        i = pl.program_id(0)
        row = i * KB + lax.broadcasted_iota(jnp.int32, (1, 1, KB), 2)
        amp_ref[...] = jnp.where(row < NF, amp[None, None, :], -1.0)

    re, amp = pl.pallas_call(
        body,
        grid=(NFP // KB,),
        in_specs=[
            pl.BlockSpec((TSEQ, D), lambda i: (0, 0)),
            pl.BlockSpec((KB, TSEQ), lambda i: (i, 0)),
            pl.BlockSpec((KB, TSEQ), lambda i: (i, 0)),
            pl.BlockSpec((KB, TSEQ), lambda i: (i, 0)),
            pl.BlockSpec((KB, TSEQ), lambda i: (i, 0)),
        ],
        out_specs=[
            pl.BlockSpec((KB, D), lambda i: (i, 0)),
            pl.BlockSpec((1, 1, KB), lambda i: (i, 0, 0)),
        ],
        out_shape=[
            jax.ShapeDtypeStruct((NFP, D), _F32),
            jax.ShapeDtypeStruct((NFP // KB, 1, KB), _F32),
        ],
    )(h, cchi, cclo, cshi, cslo)
    return re, amp


def _sc_topk_gather(amp, re, basis):
    nvreg = NFP // 16
    mesh = plsc.VectorSubcoreMesh(core_axis_name="c", subcore_axis_name="s")

    @functools.partial(
        pl.kernel,
        mesh=mesh,
        out_type=[
            jax.ShapeDtypeStruct((TOPK,), jnp.int32),
            jax.ShapeDtypeStruct((TOPK, D), _F32),
            jax.ShapeDtypeStruct((TOPK, TSEQP), _F32),
        ],
        scratch_types=[
            pltpu.VMEM((NFP,), _F32),
            pltpu.VMEM((TOPK,), jnp.int32),
            pltpu.VMEM((TOPK, D), _F32),
            pltpu.VMEM((TOPK, TSEQP), _F32),
            pltpu.SemaphoreType.DMA,
        ],
    )
    def sc_kernel(amp_hbm, re_hbm, basis_hbm, idx_out, rows_out, brows_out,
                  amp_v, idx_v, rows_v, brows_v, sem):
        wid = lax.axis_index("s") * 2 + lax.axis_index("c")

        @pl.when(wid == 0)
        def _():
            pltpu.sync_copy(amp_hbm, amp_v)
            lanes = lax.iota(jnp.int32, 16)

            def _perm(v, s):
                return v.at[jnp.bitwise_xor(lanes, s)].get(
                    mode="promise_in_bounds")

            def round_body(rnd, best_prev):
                m16 = jnp.full((16,), -3e38, _F32)
                a16 = jnp.full((16,), 2 ** 30, jnp.int32)
                for j in range(nvreg):
                    v = amp_v[pl.ds(j * 16, 16)]
                    idxv = lanes + j * 16
                    v = jnp.where(idxv == best_prev, _F32(-3e38), v)
                    amp_v[pl.ds(j * 16, 16)] = v
                    take = v > m16
                    a16 = jnp.where(take, idxv, a16)
                    m16 = jnp.maximum(m16, v)
                for s in (8, 4, 2, 1):
                    mp = _perm(m16, s)
                    ap = _perm(a16, s)
                    take = (mp > m16) | ((mp == m16) & (ap < a16))
                    a16 = jnp.where(take, ap, a16)
                    m16 = jnp.where(take, mp, m16)
                rnd16 = jnp.full((16,), 1, jnp.int32) * rnd
                for j in range(TOPK // 16):
                    cur = idx_v[pl.ds(j * 16, 16)]
                    sel = (lanes + j * 16) == rnd16
                    idx_v[pl.ds(j * 16, 16)] = jnp.where(sel, a16, cur)
                return a16

            lax.fori_loop(0, TOPK, round_body,
                          jnp.full((16,), -1, jnp.int32))
            cp1 = pltpu.async_copy(re_hbm.at[idx_v], rows_v, sem)
            cp2 = pltpu.async_copy(basis_hbm.at[idx_v], brows_v, sem)
            cp1.wait()
            cp2.wait()
            pltpu.sync_copy(idx_v, idx_out)
            pltpu.sync_copy(rows_v, rows_out)
            pltpu.sync_copy(brows_v, brows_out)

    return sc_kernel(amp, re, basis)


def _feb_apply(idx2d, r, brows, wq, bq, wk, bk, wv, bv, h):
    def body(idx_ref, r_ref, brows_ref, wq_ref, bq_ref, wk_ref, bk_ref,
             wv_ref, bv_ref, h_ref, o_ref):
        r_ = r_ref[...]
        q = _dg11(r_, wq_ref[...]) + bq_ref[...]
        k = _dg11(r_, wk_ref[...]) + bk_ref[...]
        v = _dg11(r_, wv_ref[...]) + bv_ref[...]
        s = _dg11(q, k, precision=_HI) * (1.0 / np.sqrt(D).astype(np.float32))
        s = s - s.max(axis=1, keepdims=True)
        e = jnp.exp(s)
        a = e / e.sum(axis=1, keepdims=True)
        ao = _dg10(a, v, precision=_HI)

        idx = idx_ref[0, :]
        coef = jnp.where((idx == 0) | (idx == TSEQ // 2), 1.0, 2.0) * (
            1.0 / TSEQ)
        aos = ao * coef[:, None].astype(_F32)
        fo = lax.dot_general(brows_ref[...], aos, (((0,), (0,)), ((), ())),
                             precision=_HI, preferred_element_type=_F32)
        o_ref[...] = h_ref[...] + fo[:TSEQ, :]

    return pl.pallas_call(
        body,
        out_shape=jax.ShapeDtypeStruct((TSEQ, D), _F32),
    )(idx2d, r, brows, wq, bq, wk, bk, wv, bv, h)


def _mm_bias(x, w, b, nb, relu=False):
    M, K = x.shape
    NW = w.shape[0]

    def body(x_ref, w_ref, b_ref, o_ref):
        y = _dg11(x_ref[...], w_ref[...]) + b_ref[...]
        if relu:
            y = jnp.maximum(y, 0.0)
        o_ref[...] = y

    return pl.pallas_call(
        body,
        grid=(NW // nb,),
        in_specs=[
            pl.BlockSpec((M, K), lambda j: (0, 0)),
            pl.BlockSpec((nb, K), lambda j: (j, 0)),
            pl.BlockSpec((1, nb), lambda j: (0, j)),
        ],
        out_specs=pl.BlockSpec((M, nb), lambda j: (0, j)),
        out_shape=jax.ShapeDtypeStruct((M, NW), _F32),
    )(x, w, b)


def _qkv3(x, w, b3):
    HB = 4

    def body(x_ref, w_ref, b_ref, o_ref):
        y = _dg11(x_ref[...], w_ref[...])
        for t in range(HB):
            o_ref[t] = y[:, t * DH:(t + 1) * DH] + b_ref[t]

    return pl.pallas_call(
        body,
        grid=(3 * H // HB,),
        in_specs=[
            pl.BlockSpec((TSEQ, D), lambda j: (0, 0)),
            pl.BlockSpec((HB * DH, D), lambda j: (j, 0)),
            pl.BlockSpec((HB, 1, DH), lambda j: (j, 0, 0)),
        ],
        out_specs=pl.BlockSpec((HB, TSEQ, DH), lambda j: (j, 0, 0)),
        out_shape=jax.ShapeDtypeStruct((3 * H, TSEQ, DH), _F32),
    )(x, w, b3)


def _mha_core(qkv3):
    def body(q_ref, k_ref, v_ref, o_ref):
        outs = []
        for hh in range(2):
            s = _dg11(q_ref[hh], k_ref[hh]) * (
                1.0 / np.sqrt(DH).astype(np.float32))
            e = jnp.exp(s)
            recip = 1.0 / e.sum(axis=1, keepdims=True)
            outs.append(_dg10(e, v_ref[hh]) * recip)
        o_ref[...] = jnp.concatenate(outs, axis=1)

    return pl.pallas_call(
        body,
        grid=(H // 2, TSEQ // RB),
        in_specs=[
            pl.BlockSpec((2, RB, DH), lambda p, i: (p, i, 0)),
            pl.BlockSpec((2, TSEQ, DH), lambda p, i: (H // 2 + p, 0, 0)),
            pl.BlockSpec((2, TSEQ, DH), lambda p, i: (H + p, 0, 0)),
        ],
        out_specs=pl.BlockSpec((RB, 2 * DH), lambda p, i: (i, p)),
        out_shape=jax.ShapeDtypeStruct((TSEQ, D), _F32),
    )(qkv3, qkv3, qkv3)


def _ln(y, g, b):
    mu = y.mean(axis=1, keepdims=True)
    ms = (y * y).mean(axis=1, keepdims=True)
    var = ms - mu * mu
    k = 1.0 / jnp.sqrt(var + 1e-5)
    return (y - mu) * k * g + b


def _encoder_tail(o, wout, bout, res, g1, bn1, w1, b1, w2, b2, g2, bn2):
    def body(o_ref, wout_ref, bout_ref, res_ref, g1_ref, bn1_ref,
             w1_ref, b1_ref, w2_ref, b2_ref, g2_ref, bn2_ref, y_ref):
        x1 = _ln(_dg11(o_ref[...], wout_ref[...]) + bout_ref[...]
                 + res_ref[...], g1_ref[...], bn1_ref[...])
        a = jnp.maximum(_dg11(x1, w1_ref[...]) + b1_ref[...], 0.0)
        y = _dg11(a, w2_ref[...]) + b2_ref[...] + x1
        y_ref[...] = _ln(y, g2_ref[...], bn2_ref[...])

    vec = pl.BlockSpec((1, D), lambda i: (0, 0))
    return pl.pallas_call(
        body,
        grid=(TSEQ // RB,),
        in_specs=[
            pl.BlockSpec((RB, D), lambda i: (i, 0)),
            pl.BlockSpec((D, D), lambda i: (0, 0)),
            vec,
            pl.BlockSpec((RB, D), lambda i: (i, 0)),
            vec, vec,
            pl.BlockSpec((DFF, D), lambda i: (0, 0)),
            pl.BlockSpec((1, DFF), lambda i: (0, 0)),
            pl.BlockSpec((D, DFF), lambda i: (0, 0)),
            vec, vec, vec,
        ],
        out_specs=pl.BlockSpec((RB, D), lambda i: (i, 0)),
        out_shape=jax.ShapeDtypeStruct((TSEQ, D), _F32),
    )(o, wout, bout, res, g1, bn1, w1, b1, w2, b2, g2, bn2)


def _pool_fc(h, w, b):
    def body(h_ref, w_ref, b_ref, o_ref):
        pooled = h_ref[...].mean(axis=0, keepdims=True)
        o_ref[...] = _dg11(pooled, w_ref[...], precision=_HI) + b_ref[...]

    nout = w.shape[0]
    return pl.pallas_call(
        body,
        out_shape=jax.ShapeDtypeStruct((1, nout), _F32),
    )(h, w, b)


def kernel(x, params):
    cchi = jnp.asarray(_CCOS_HI)
    cclo = jnp.asarray(_CCOS_LO)
    cshi = jnp.asarray(_CSIN_HI)
    cslo = jnp.asarray(_CSIN_LO)
    pe = jnp.asarray(_PE_NP)
    basis = jnp.asarray(_BASIS_NP)

    x2 = x[0]
    xp = jnp.pad(x2, ((2 * PADC, 2 * PADC), (0, 0)))
    h = _embed(xp, params['in_w'], params['in_b'][None], pe)

    for lp in params['layers']:
        re, amp = _spectrum(h, cchi, cclo, cshi, cslo)
        idx, r, brows = _sc_topk_gather(amp.reshape(NFP), re, basis)
        h = _feb_apply(idx.reshape(1, TOPK), r, brows, lp['wq'],
                       lp['bq'][None], lp['wk'], lp['bk'][None],
                       lp['wv'], lp['bv'][None], h)

        qkv3 = _qkv3(h, lp['win'], lp['bin'].reshape(3 * H, 1, DH))
        o = _mha_core(qkv3)
        h = _encoder_tail(o, lp['wout'], lp['bout'][None], h,
                          lp['n1g'][None], lp['n1b'][None],
                          lp['w1'], lp['b1'][None], lp['w2'], lp['b2'][None],
                          lp['n2g'][None], lp['n2b'][None])

    out = _pool_fc(h, params['fc_w'], params['fc_b'][None])
    return out.reshape(1, 96, 8)
